# Initial kernel scaffold; baseline (speedup 1.0000x reference)
import jax
import jax.numpy as jnp
from jax.experimental import pallas as pl


def kernel(inputs, targets_, targets, GT_MC):
    raise NotImplementedError("write your pallas kernel here")


# speedup vs baseline: 25.9457x; 25.9457x over previous
import functools

import jax
import jax.numpy as jnp
from jax.experimental import pallas as pl

DELTA = 5.0
RFRAC = 0.01
BLOCK_M = 32


def _mmcl_body(x_ref, t_ref, out_ref, *, k):
    x = x_ref[...]
    t = t_ref[...]
    col = jax.lax.broadcasted_iota(jnp.int32, x.shape, 1)
    is_pos = col == t
    pos = jnp.sum(jnp.where(is_pos, x, 0.0), axis=1, keepdims=True)
    xm = jnp.where(is_pos, jnp.float32(-1e9), x)
    bits = jax.lax.bitcast_convert_type(xm, jnp.int32)
    skey = bits ^ ((bits >> 31) & jnp.int32(0x7FFFFFFF))

    def body(i, prefix):
        bitmask = jnp.int32(1) << (31 - i)
        cand = prefix ^ bitmask
        cnt = jnp.sum((skey >= cand).astype(jnp.int32), axis=1, keepdims=True)
        return jnp.where(cnt >= k, cand, prefix)

    prefix0 = jnp.full((x.shape[0], 1), jnp.int32(-(2 ** 31)))
    tkey = jax.lax.fori_loop(0, 32, body, prefix0)
    gt = skey > tkey
    cnt_gt = jnp.sum(gt.astype(jnp.int32), axis=1, keepdims=True)
    sum_gt = jnp.sum(jnp.where(gt, (1.0 + xm) ** 2, 0.0), axis=1, keepdims=True)
    tbits = tkey ^ ((tkey >> 31) & jnp.int32(0x7FFFFFFF))
    tval = jax.lax.bitcast_convert_type(tbits, jnp.float32)
    neg = (sum_gt + (k - cnt_gt).astype(jnp.float32) * (1.0 + tval) ** 2) * (1.0 / k)
    out_ref[...] = DELTA * (1.0 - pos) ** 2 + neg


def kernel(inputs, targets_, targets, GT_MC):
    m, n = inputs.shape
    k = int(RFRAC * (n - 1))
    t2 = targets.astype(jnp.int32)[:, None]
    grid = m // BLOCK_M
    out = pl.pallas_call(
        functools.partial(_mmcl_body, k=k),
        grid=(grid,),
        in_specs=[
            pl.BlockSpec((BLOCK_M, n), lambda i: (i, 0)),
            pl.BlockSpec((BLOCK_M, 1), lambda i: (i, 0)),
        ],
        out_specs=pl.BlockSpec((BLOCK_M, 1), lambda i: (i, 0)),
        out_shape=jax.ShapeDtypeStruct((m, 1), jnp.float32),
    )(inputs, t2)
    return jnp.mean(out)


# speedup vs baseline: 36.2551x; 1.3973x over previous
import functools

import jax
import jax.numpy as jnp
from jax.experimental import pallas as pl
from jax.experimental.pallas import tpu as pltpu

DELTA = 5.0
RFRAC = 0.01
BLOCK_M = 128


def _mmcl_body(x_ref, t_ref, out_ref, *, k):
    x = x_ref[...]
    t = t_ref[...]
    col = jax.lax.broadcasted_iota(jnp.int32, x.shape, 1)
    is_pos = col == t
    pos = jnp.sum(jnp.where(is_pos, x, 0.0), axis=1, keepdims=True)
    xm = jnp.where(is_pos, jnp.float32(-1e9), x)
    bits = jax.lax.bitcast_convert_type(xm, jnp.int32)
    skey = bits ^ ((bits >> 31) & jnp.int32(0x7FFFFFFF))

    def body(i, prefix):
        bitmask = jnp.int32(1) << (31 - i)
        cand = prefix ^ bitmask
        cnt = jnp.sum((skey >= cand).astype(jnp.int32), axis=1, keepdims=True)
        return jnp.where(cnt >= k, cand, prefix)

    prefix0 = jnp.full((x.shape[0], 1), jnp.int32(-(2 ** 31)))
    tkey = jax.lax.fori_loop(0, 32, body, prefix0)
    gt = skey > tkey
    cnt_gt = jnp.sum(gt.astype(jnp.int32), axis=1, keepdims=True)
    sum_gt = jnp.sum(jnp.where(gt, (1.0 + xm) ** 2, 0.0), axis=1, keepdims=True)
    tbits = tkey ^ ((tkey >> 31) & jnp.int32(0x7FFFFFFF))
    tval = jax.lax.bitcast_convert_type(tbits, jnp.float32)
    neg = (sum_gt + (k - cnt_gt).astype(jnp.float32) * (1.0 + tval) ** 2) * (1.0 / k)
    out_ref[...] = DELTA * (1.0 - pos) ** 2 + neg


def kernel(inputs, targets_, targets, GT_MC):
    m, n = inputs.shape
    k = int(RFRAC * (n - 1))
    t2 = targets.astype(jnp.int32)[:, None]
    grid = m // BLOCK_M
    out = pl.pallas_call(
        functools.partial(_mmcl_body, k=k),
        grid=(grid,),
        in_specs=[
            pl.BlockSpec((BLOCK_M, n), lambda i: (i, 0)),
            pl.BlockSpec((BLOCK_M, 1), lambda i: (i, 0)),
        ],
        out_specs=pl.BlockSpec((BLOCK_M, 1), lambda i: (i, 0)),
        out_shape=jax.ShapeDtypeStruct((m, 1), jnp.float32),
        compiler_params=pltpu.CompilerParams(
            dimension_semantics=("parallel",)),
    )(inputs, t2)
    return jnp.mean(out)
